# baseline (speedup 1.0000x reference)
import jax
import jax.numpy as jnp
from jax import lax
from jax.experimental import pallas as pl
from jax.experimental.pallas import tpu as pltpu
from jax.experimental.pallas import tpu_sc as plsc

_N = 65536
_H = _N // 2
_W = 1280
_OW = 256

_M2 = 12288
_K2 = _H - _M2
_K = 2 * _K2

_NC = 2
_NS = 16
_RPW = _K2 // _NS
_CB = 32
_NCHUNK = _RPW // _CB

_TB = 1024
_MB = 2048


def _sc_body(x_hbm, o_hbm, in_v, out_v):
    c = lax.axis_index("c")
    s = lax.axis_index("s")
    wid = s * _NC + c
    half = wid // _NS
    widh = wid % _NS
    in_base = half * _H + _M2 + widh * _RPW
    out_base = half * _K2 + widh * _RPW

    def chunk(i, carry):
        pltpu.sync_copy(x_hbm.at[pl.ds(in_base + i * _CB, _CB)], in_v)

        def row(b, carry2):
            for f in range(128 // 16):
                for seg in range(2):
                    p = in_v[b, pl.ds(seg * 640 + f * 16, 16)]
                    for r in range(1, 5):
                        p = p * in_v[b, pl.ds(seg * 640 + r * 128 + f * 16, 16)]
                    out_v[b, pl.ds(seg * 128 + f * 16, 16)] = p
            return carry2

        lax.fori_loop(0, _CB, row, 0)
        pltpu.sync_copy(out_v, o_hbm.at[pl.ds(out_base + i * _CB, _CB)])
        return carry

    lax.fori_loop(0, _NCHUNK, chunk, 0)


def _sc_call(x2):
    mesh = plsc.VectorSubcoreMesh(core_axis_name="c", subcore_axis_name="s")
    f = pl.kernel(
        _sc_body,
        mesh=mesh,
        out_type=jax.ShapeDtypeStruct((_K, _OW), jnp.float32),
        scratch_types=[
            pltpu.VMEM((_CB, _W), jnp.float32),
            pltpu.VMEM((_CB, _OW), jnp.float32),
        ],
    )
    return f(x2)


def _tc_head_body(a_ref, b_ref, o_ref):
    for h, ref in enumerate((a_ref, b_ref)):
        x = ref[0]
        o_ref[h, :, 0:128] = (x[:, 0:128] * x[:, 128:256] * x[:, 256:384]
                              * x[:, 384:512] * x[:, 512:640])
        o_ref[h, :, 128:256] = (x[:, 640:768] * x[:, 768:896] * x[:, 896:1024]
                                * x[:, 1024:1152] * x[:, 1152:1280])


def _tc_head(x3):
    return pl.pallas_call(
        _tc_head_body,
        grid=(_M2 // _TB,),
        in_specs=[
            pl.BlockSpec((1, _TB, _W), lambda i: (0, i, 0)),
            pl.BlockSpec((1, _TB, _W), lambda i: (1, i, 0)),
        ],
        out_specs=pl.BlockSpec((2, _TB, _OW), lambda i: (0, i, 0)),
        out_shape=jax.ShapeDtypeStruct((2, _H, _OW), jnp.float32),
    )(x3, x3)


def _tc_merge_body(sc_ref, f_ref, o_ref):
    o_ref[...] = sc_ref[...].reshape(1, _MB, _OW)


def _tc_merge(out_sc, head):
    nb = _K2 // _MB
    return pl.pallas_call(
        _tc_merge_body,
        grid=(_K // _MB,),
        in_specs=[
            pl.BlockSpec((_MB, _OW), lambda j: (j, 0)),
            pl.BlockSpec((1, 8, _OW), lambda j: (0, 0, 0)),
        ],
        out_specs=pl.BlockSpec(
            (1, _MB, _OW), lambda j: (j // nb, _M2 // _MB + j % nb, 0)),
        out_shape=jax.ShapeDtypeStruct((2, _H, _OW), jnp.float32),
        input_output_aliases={1: 0},
    )(out_sc, head)


def kernel(inputs):
    n, r, d = inputs.shape
    x2 = inputs.reshape(n, r * d)
    x3 = inputs.reshape(2, _H, r * d)
    out_sc = _sc_call(x2)
    head = _tc_head(x3)
    out2 = _tc_merge(out_sc, head)
    return out2.reshape(n, 2, d)


# speedup vs baseline: 4.5604x; 4.5604x over previous
import jax
import jax.numpy as jnp
from jax.experimental import pallas as pl

_B = 1024


def _prods(x):
    p0 = x[:, 0, :] * x[:, 1, :] * x[:, 2, :] * x[:, 3, :] * x[:, 4, :]
    p1 = x[:, 5, :] * x[:, 6, :] * x[:, 7, :] * x[:, 8, :] * x[:, 9, :]
    return jnp.stack([p0, p1], axis=1)


def _body(a_ref, b_ref, o_ref):
    o_ref[0] = _prods(a_ref[0])
    o_ref[1] = _prods(b_ref[0])


def kernel(inputs):
    n, r, d = inputs.shape
    h = n // 2
    x = inputs.reshape(2, h, r, d)
    out = pl.pallas_call(
        _body,
        grid=(h // _B,),
        in_specs=[
            pl.BlockSpec((1, _B, r, d), lambda i: (0, i, 0, 0)),
            pl.BlockSpec((1, _B, r, d), lambda i: (1, i, 0, 0)),
        ],
        out_specs=pl.BlockSpec((2, _B, 2, d), lambda i: (0, i, 0, 0)),
        out_shape=jax.ShapeDtypeStruct((2, h, 2, d), inputs.dtype),
    )(x, x)
    return out.reshape(n, 2, d)
